# baseline (speedup 1.0000x reference)
import functools

import jax
import jax.numpy as jnp
from jax.experimental import pallas as pl

TOP_P = 0.8
PAD_N = 100352
ROWS2D = 784


def _topp_sample_body(sp_ref, cum_ref, sidx_ref, g_ref, out_ref, tok_ref):
    p = sp_ref[0]
    cum = cum_ref[0]
    sidx = sidx_ref[0]
    g = g_ref[0]

    keep = (cum - p) <= TOP_P
    pk = jnp.where(keep, p, 0.0)
    s = jnp.sum(pk)
    out = pk / s
    out_ref[0] = out

    t = jnp.log(out + 1e-20) + g
    m = jnp.max(t)
    pos = jax.lax.broadcasted_iota(jnp.int32, t.shape, 0) * 128 + \
        jax.lax.broadcasted_iota(jnp.int32, t.shape, 1)
    big = jnp.int32(2**31 - 1)
    jmin = jnp.min(jnp.where(t == m, pos, big))
    tok = jnp.max(jnp.where(pos == jmin, sidx, jnp.int32(-1)))
    tok_ref[0] = jnp.full((1, 128), tok, jnp.int32)


def kernel(logits):
    b, n = logits.shape
    probs = jax.nn.softmax(logits, axis=-1)
    sorted_indices = jnp.argsort(-probs, axis=-1)
    sorted_probs = jnp.take_along_axis(probs, sorted_indices, axis=-1)
    cum = jnp.cumsum(sorted_probs, axis=-1)
    g = jax.random.gumbel(jax.random.key(42), (b, n), jnp.float32)

    pad = PAD_N - n
    sp = jnp.pad(sorted_probs, ((0, 0), (0, pad))).reshape(b, ROWS2D, 128)
    cm = jnp.pad(cum, ((0, 0), (0, pad)), constant_values=2.0).reshape(
        b, ROWS2D, 128)
    si = jnp.pad(sorted_indices, ((0, 0), (0, pad))).reshape(b, ROWS2D, 128)
    gg = jnp.pad(g, ((0, 0), (0, pad))).reshape(b, ROWS2D, 128)

    out, tok = pl.pallas_call(
        _topp_sample_body,
        grid=(b,),
        in_specs=[
            pl.BlockSpec((1, ROWS2D, 128), lambda i: (i, 0, 0)),
            pl.BlockSpec((1, ROWS2D, 128), lambda i: (i, 0, 0)),
            pl.BlockSpec((1, ROWS2D, 128), lambda i: (i, 0, 0)),
            pl.BlockSpec((1, ROWS2D, 128), lambda i: (i, 0, 0)),
        ],
        out_specs=[
            pl.BlockSpec((1, ROWS2D, 128), lambda i: (i, 0, 0)),
            pl.BlockSpec((1, 1, 128), lambda i: (i, 0, 0)),
        ],
        out_shape=[
            jax.ShapeDtypeStruct((b, ROWS2D, 128), jnp.float32),
            jax.ShapeDtypeStruct((b, 1, 128), jnp.int32),
        ],
    )(sp, cm, si, gg)

    sorted_probs_out = out.reshape(b, PAD_N)[:, :n]
    return (tok[:, 0, :1], sorted_probs_out)
